# baseline (speedup 1.0000x reference)
import functools

import jax
import jax.numpy as jnp
from jax import lax
from jax.experimental import pallas as pl
from jax.experimental.pallas import tpu as pltpu
from jax.experimental.pallas import tpu_sc as plsc

MAX_POS = 100000
HALF = 32
EMB = 64
SEQ = 199
S1 = SEQ + 1
ROW = S1 * EMB
L = 16
NVEC = 13


def _make_pe_add(B, n_workers):
    rows_per_w = B // n_workers
    mesh = plsc.VectorSubcoreMesh(core_axis_name="c", subcore_axis_name="s")

    @functools.partial(
        pl.kernel,
        mesh=mesh,
        compiler_params=pltpu.CompilerParams(use_tc_tiling_on_sc=False),
        out_type=jax.ShapeDtypeStruct((B * ROW,), jnp.float32),
        scratch_types=[
            pltpu.VMEM((NVEC * L,), jnp.float32),
            pltpu.VMEM((NVEC * L,), jnp.int32),
            pltpu.VMEM((ROW,), jnp.float32),
            pltpu.VMEM((S1, HALF), jnp.float32),
            pltpu.VMEM((S1, HALF), jnp.float32),
            pltpu.SemaphoreType.DMA,
            pltpu.SemaphoreType.DMA,
        ],
    )
    def pe_add(x_hbm, px_hbm, pex_hbm, pey_hbm, out_hbm,
               px_v, idx_v, xrow_v, pex_v, pey_v, sem_x, sem_g):
        cid = lax.axis_index("c")
        sid = lax.axis_index("s")
        wid = sid * 2 + cid
        base = wid * rows_per_w

        pltpu.sync_copy(pex_hbm.at[0], pex_v.at[0])
        pltpu.sync_copy(pey_hbm.at[0], pey_v.at[0])

        def row_body(i, carry):
            b = base + i
            cp_x = pltpu.make_async_copy(
                x_hbm.at[pl.ds(b * ROW, ROW)], xrow_v, sem_x)
            cp_x.start()
            pltpu.sync_copy(px_hbm.at[pl.ds(b * S1, S1)],
                            px_v.at[pl.ds(0, S1)])
            for k in range(NVEC):
                v = px_v[pl.ds(k * L, L)]
                q = v / 20.0
                t = q.astype(jnp.int32)
                c = jnp.where(v > t.astype(jnp.float32) * 20.0, t + 1, t)
                idx_v[pl.ds(k * L, L)] = c
            cp_g0 = pltpu.make_async_copy(
                pex_hbm.at[idx_v.at[pl.ds(0, 128)]],
                pex_v.at[pl.ds(1, 128)], sem_g)
            cp_g1 = pltpu.make_async_copy(
                pex_hbm.at[idx_v.at[pl.ds(128, SEQ - 128)]],
                pex_v.at[pl.ds(129, SEQ - 128)], sem_g)
            cp_g2 = pltpu.make_async_copy(
                pey_hbm.at[idx_v.at[pl.ds(0, 128)]],
                pey_v.at[pl.ds(1, 128)], sem_g)
            cp_g3 = pltpu.make_async_copy(
                pey_hbm.at[idx_v.at[pl.ds(128, SEQ - 128)]],
                pey_v.at[pl.ds(129, SEQ - 128)], sem_g)
            cp_g0.start()
            cp_g1.start()
            cp_g2.start()
            cp_g3.start()
            cp_g0.wait()
            cp_g1.wait()
            cp_g2.wait()
            cp_g3.wait()
            cp_x.wait()

            def add_body(t, c2):
                o = t * EMB
                xa = xrow_v[pl.ds(o, L)]
                xb = xrow_v[pl.ds(o + L, L)]
                xc = xrow_v[pl.ds(o + 2 * L, L)]
                xd = xrow_v[pl.ds(o + 3 * L, L)]
                xrow_v[pl.ds(o, L)] = xa + pex_v[t, pl.ds(0, L)]
                xrow_v[pl.ds(o + L, L)] = xb + pex_v[t, pl.ds(L, L)]
                xrow_v[pl.ds(o + 2 * L, L)] = xc + pey_v[t, pl.ds(0, L)]
                xrow_v[pl.ds(o + 3 * L, L)] = xd + pey_v[t, pl.ds(L, L)]
                return c2

            lax.fori_loop(0, S1, add_body, 0)
            pltpu.sync_copy(xrow_v, out_hbm.at[pl.ds(b * ROW, ROW)])
            return carry

        lax.fori_loop(0, rows_per_w, row_body, 0)

    return pe_add


def kernel(x, px, py, pex_w, pey_w):
    del py
    B = x.shape[0]
    info = plsc.get_sparse_core_info()
    n_workers = info.num_cores * info.num_subcores
    x_f = x.reshape(B * ROW)
    px_p = jnp.pad(px, ((0, 0), (0, S1 - SEQ))).reshape(B * S1)
    pe_add = _make_pe_add(B, n_workers)
    out = pe_add(x_f, px_p, pex_w, pey_w)
    return out.reshape(B, S1, EMB)


# speedup vs baseline: 1.1388x; 1.1388x over previous
import functools

import jax
import jax.numpy as jnp
from jax import lax
from jax.experimental import pallas as pl
from jax.experimental.pallas import tpu as pltpu
from jax.experimental.pallas import tpu_sc as plsc

MAX_POS = 100000
HALF = 32
EMB = 64
SEQ = 199
S1 = SEQ + 1
ROW = S1 * EMB
L = 16
NVEC = 13
PXW = NVEC * L


def _make_pe_add(B, n_workers):
    rows_per_w = B // n_workers
    half_iters = rows_per_w // 2
    mesh = plsc.VectorSubcoreMesh(core_axis_name="c", subcore_axis_name="s")

    @functools.partial(
        pl.kernel,
        mesh=mesh,
        compiler_params=pltpu.CompilerParams(use_tc_tiling_on_sc=False),
        out_type=jax.ShapeDtypeStruct((B * ROW,), jnp.float32),
        scratch_types=[
            pltpu.VMEM((rows_per_w * PXW,), jnp.float32),
            pltpu.VMEM((NVEC * L,), jnp.int32),
            pltpu.VMEM((NVEC * L,), jnp.int32),
            pltpu.VMEM((ROW,), jnp.float32),
            pltpu.VMEM((ROW,), jnp.float32),
            pltpu.VMEM((S1, HALF), jnp.float32),
            pltpu.VMEM((S1, HALF), jnp.float32),
            pltpu.VMEM((S1, HALF), jnp.float32),
            pltpu.VMEM((S1, HALF), jnp.float32),
            pltpu.SemaphoreType.DMA,
            pltpu.SemaphoreType.DMA,
            pltpu.SemaphoreType.DMA,
            pltpu.SemaphoreType.DMA,
            pltpu.SemaphoreType.DMA,
            pltpu.SemaphoreType.DMA,
        ],
    )
    def pe_add(x_hbm, px_hbm, pex_hbm, pey_hbm, out_hbm,
               pxall_v, idx_a, idx_b, xrow_a, xrow_b,
               pex_a, pex_b, pey_a, pey_b,
               sem_xa, sem_xb, sem_ga, sem_gb, sem_oa, sem_ob):
        cid = lax.axis_index("c")
        sid = lax.axis_index("s")
        wid = sid * 2 + cid
        base = wid * rows_per_w

        pltpu.sync_copy(px_hbm.at[pl.ds(base * PXW, rows_per_w * PXW)],
                        pxall_v)
        pltpu.sync_copy(pex_hbm.at[0], pex_a.at[0])
        pltpu.sync_copy(pex_hbm.at[0], pex_b.at[0])
        pltpu.sync_copy(pey_hbm.at[0], pey_a.at[0])
        pltpu.sync_copy(pey_hbm.at[0], pey_b.at[0])

        def issue_row(r, il, idx_v, xrow_v, pex_v, pey_v, sem_x, sem_g):
            cp_x = pltpu.make_async_copy(
                x_hbm.at[pl.ds(r * ROW, ROW)], xrow_v, sem_x)
            cp_x.start()
            for k in range(NVEC):
                v = pxall_v[pl.ds(il * PXW + k * L, L)]
                q = v / 20.0
                t = q.astype(jnp.int32)
                c = jnp.where(v > t.astype(jnp.float32) * 20.0, t + 1, t)
                idx_v[pl.ds(k * L, L)] = c
            cps = [
                pltpu.make_async_copy(
                    pex_hbm.at[idx_v.at[pl.ds(0, 128)]],
                    pex_v.at[pl.ds(1, 128)], sem_g),
                pltpu.make_async_copy(
                    pex_hbm.at[idx_v.at[pl.ds(128, SEQ - 128)]],
                    pex_v.at[pl.ds(129, SEQ - 128)], sem_g),
                pltpu.make_async_copy(
                    pey_hbm.at[idx_v.at[pl.ds(0, 128)]],
                    pey_v.at[pl.ds(1, 128)], sem_g),
                pltpu.make_async_copy(
                    pey_hbm.at[idx_v.at[pl.ds(128, SEQ - 128)]],
                    pey_v.at[pl.ds(129, SEQ - 128)], sem_g),
            ]
            for cp in cps:
                cp.start()
            return (cp_x, cps)

        def finish_row(r, handles, xrow_v, pex_v, pey_v, sem_o):
            cp_x, cps = handles
            for cp in cps:
                cp.wait()
            cp_x.wait()

            def add_body(t, c2):
                for tt in (2 * t, 2 * t + 1):
                    o = tt * EMB
                    plsc.addupdate(xrow_v.at[pl.ds(o, L)],
                                   pex_v[tt, pl.ds(0, L)])
                    plsc.addupdate(xrow_v.at[pl.ds(o + L, L)],
                                   pex_v[tt, pl.ds(L, L)])
                    plsc.addupdate(xrow_v.at[pl.ds(o + 2 * L, L)],
                                   pey_v[tt, pl.ds(0, L)])
                    plsc.addupdate(xrow_v.at[pl.ds(o + 3 * L, L)],
                                   pey_v[tt, pl.ds(L, L)])
                return c2

            lax.fori_loop(0, S1 // 2, add_body, 0)
            cp_o = pltpu.make_async_copy(
                xrow_v, out_hbm.at[pl.ds(r * ROW, ROW)], sem_o)
            cp_o.start()

        def pair_body(j, carry):
            r0 = base + 2 * j
            r1 = r0 + 1

            @pl.when(j > 0)
            def _():
                pltpu.make_async_copy(
                    xrow_a, out_hbm.at[pl.ds(r0 * ROW, ROW)], sem_oa).wait()
                pltpu.make_async_copy(
                    xrow_b, out_hbm.at[pl.ds(r1 * ROW, ROW)], sem_ob).wait()

            ha = issue_row(r0, 2 * j, idx_a, xrow_a, pex_a, pey_a,
                           sem_xa, sem_ga)
            hb = issue_row(r1, 2 * j + 1, idx_b, xrow_b, pex_b, pey_b,
                           sem_xb, sem_gb)
            finish_row(r0, ha, xrow_a, pex_a, pey_a, sem_oa)
            finish_row(r1, hb, xrow_b, pex_b, pey_b, sem_ob)
            return carry

        lax.fori_loop(0, half_iters, pair_body, 0)
        rl0 = base + rows_per_w - 2
        pltpu.make_async_copy(
            xrow_a, out_hbm.at[pl.ds(rl0 * ROW, ROW)], sem_oa).wait()
        pltpu.make_async_copy(
            xrow_b, out_hbm.at[pl.ds((rl0 + 1) * ROW, ROW)], sem_ob).wait()

    return pe_add


def kernel(x, px, py, pex_w, pey_w):
    del py
    B = x.shape[0]
    info = plsc.get_sparse_core_info()
    n_workers = info.num_cores * info.num_subcores
    x_f = x.reshape(B * ROW)
    px_p = jnp.pad(px, ((0, 0), (0, PXW - SEQ))).reshape(B * PXW)
    pe_add = _make_pe_add(B, n_workers)
    out = pe_add(x_f, px_p, pex_w, pey_w)
    return out.reshape(B, S1, EMB)
